# baseline (device time: 73465 ns/iter reference)
import jax
import jax.numpy as jnp
from jax import lax
from jax.experimental import pallas as pl
from jax.experimental.pallas import tpu as pltpu

B, H, D, BS = 8, 8, 128, 16
NB = 512
HL = 2
CP = 64
CK = CP * BS
NSLOT = 4

NEG_INF = -1e30


def kernel(Q, K, V, bt, lens):
    npages = K.shape[0]
    nkeys = npages * BS
    nchunks = npages // CP

    my_x_o = lax.axis_index("x")
    my_z_o = lax.axis_index("z")
    idx = my_x_o * 2 + my_z_o
    hb = idx * HL

    Kr = K.reshape(nkeys, H * D)
    Vr = V.reshape(nkeys, H * D)
    q_my = lax.dynamic_slice_in_dim(Q.reshape(B, H, D), hb, HL, axis=1)

    def body(q_ref, k_hbm, v_hbm, bt_ref, lens_ref, out_ref,
             comm_ref, gbuf, kbuf, vbuf, ksem, vsem, gsem,
             send_sem, recv_sem, ag_send, ag_recv):
        my_x = lax.axis_index("x")
        my_y = lax.axis_index("y")
        my_z = lax.axis_index("z")
        my_idx = my_x * 2 + my_z
        col0 = (my_idx * HL) * D

        y_partner = (my_x, 1 - my_y, my_z)
        ag_partners = [
            ((1 - my_x, my_y, my_z), (1 - my_x) * 2 + my_z, 0),
            ((my_x, my_y, 1 - my_z), my_x * 2 + (1 - my_z), 1),
            ((1 - my_x, my_y, 1 - my_z), (1 - my_x) * 2 + (1 - my_z), 2),
        ]

        barrier = pltpu.get_barrier_semaphore()
        for dev in [y_partner] + [p[0] for p in ag_partners]:
            pl.semaphore_signal(barrier, inc=1, device_id=dev,
                                device_id_type=pl.DeviceIdType.MESH)
        pl.semaphore_wait(barrier, 4)

        def start_copy(t):
            slot = t % NSLOT
            kc = pltpu.make_async_copy(
                k_hbm.at[t * CK:(t + 1) * CK, pl.ds(col0, HL * D)],
                kbuf.at[slot], ksem.at[slot])
            vc = pltpu.make_async_copy(
                v_hbm.at[t * CK:(t + 1) * CK, pl.ds(col0, HL * D)],
                vbuf.at[slot], vsem.at[slot])
            kc.start()
            vc.start()
            return kc, vc

        copies = [start_copy(t) for t in range(NSLOT - 1)]

        page_off = my_y * npages
        bt_v = bt_ref[:, :]
        lens_v = lens_ref[:, :]
        slot_i = lax.broadcasted_iota(jnp.int32, (B, NB, 1), 1)
        valid = slot_i < lens_v[:, :, None]
        pids = lax.broadcasted_iota(jnp.int32, (B, NB, npages), 2) + page_off
        match = jnp.logical_and(bt_v[:, :, None] == pids, valid)
        counts = jnp.sum(match.astype(jnp.float32), axis=1)
        countsT = counts.T
        wT = jnp.broadcast_to(countsT[:, None, :],
                              (npages, BS, B)).reshape(nkeys, B)

        qt = jnp.transpose(q_ref[:, :, :], (1, 2, 0))
        e1 = lax.broadcasted_iota(jnp.int32, (HL, HL), 0)
        e2 = lax.broadcasted_iota(jnp.int32, (HL, HL), 1)
        eye = (e1 == e2).astype(jnp.float32)
        qbd = (qt[:, :, None, :] * eye[:, None, :, None]
               ).reshape(HL * D, HL * B)

        scale = D ** -0.5
        m = None
        l = None
        acc = None
        for t in range(nchunks):
            if t + NSLOT - 1 < nchunks:
                copies.append(start_copy(t + NSLOT - 1))
            slot = t % NSLOT
            kc, vc = copies[t]
            kc.wait()
            vc.wait()
            kh = kbuf[slot]
            vh = vbuf[slot]
            wc = wT[t * CK:(t + 1) * CK]
            wc2 = jnp.concatenate([wc] * HL, axis=1)
            s = lax.dot_general(
                kh, qbd, (((1,), (0,)), ((), ())),
                preferred_element_type=jnp.float32) * scale
            m_c = jnp.max(s, axis=0, keepdims=True)
            if m is None:
                m_new = m_c
                alpha = None
            else:
                m_new = jnp.maximum(m, m_c)
                alpha = jnp.exp(m - m_new)
            e = jnp.exp(s - m_new) * wc2
            pv = lax.dot_general(
                vh, e, (((0,), (0,)), ((), ())),
                preferred_element_type=jnp.float32)
            l_c = jnp.sum(e, axis=0, keepdims=True)
            if alpha is None:
                acc = pv
                l = l_c
            else:
                acc = alpha * acc + pv
                l = alpha * l + l_c
            m = m_new

        packed = jnp.stack(
            [jnp.concatenate(
                [acc[j * D:(j + 1) * D, j * B:(j + 1) * B],
                 m[:, j * B:(j + 1) * B],
                 l[:, j * B:(j + 1) * B]], axis=0)
             for j in range(HL)], axis=0)
        comm_ref[0] = packed

        rdma = pltpu.make_async_remote_copy(
            src_ref=comm_ref.at[0],
            dst_ref=comm_ref.at[1],
            send_sem=send_sem,
            recv_sem=recv_sem,
            device_id=y_partner,
            device_id_type=pl.DeviceIdType.MESH,
        )
        rdma.start()
        rdma.wait()

        r = comm_ref[1]
        o_heads = []
        for j in range(HL):
            acc1 = packed[j, :D]
            m1 = packed[j, D:D + 1]
            l1 = packed[j, D + 1:D + 2]
            acc2 = r[j, :D]
            m2 = r[j, D:D + 1]
            l2 = r[j, D + 1:D + 2]
            m12 = jnp.maximum(m1, m2)
            a1 = jnp.exp(m1 - m12)
            a2 = jnp.exp(m2 - m12)
            lsum = a1 * l1 + a2 * l2
            o_heads.append((a1 * acc1 + a2 * acc2) / lsum)
        o_my = jnp.stack(o_heads, axis=0)

        local = pltpu.make_async_copy(comm_ref.at[2, :, :D],
                                      gbuf.at[my_idx], gsem)
        comm_ref[2, :, :D] = o_my
        local.start()
        sends = []
        recvs = []
        for dev, pidx, ds in ag_partners:
            snd = pltpu.make_async_remote_copy(
                src_ref=comm_ref.at[2, :, :D],
                dst_ref=gbuf.at[my_idx],
                send_sem=ag_send.at[ds],
                recv_sem=ag_recv.at[ds],
                device_id=dev,
                device_id_type=pl.DeviceIdType.MESH,
            )
            snd.start()
            sends.append(snd)
            recvs.append(pltpu.make_async_remote_copy(
                src_ref=comm_ref.at[2, :, :D],
                dst_ref=gbuf.at[pidx],
                send_sem=ag_send.at[ds],
                recv_sem=ag_recv.at[ds],
                device_id=dev,
                device_id_type=pl.DeviceIdType.MESH,
            ))
        local.wait()
        for snd in sends:
            snd.wait_send()
        for rcv in recvs:
            rcv.wait_recv()

        g = gbuf[...]
        out_ref[:, :, :, :] = jnp.transpose(
            g, (3, 0, 1, 2)).reshape(B, H, D)[:, None, :, :]

    return pl.pallas_call(
        body,
        out_shape=jax.ShapeDtypeStruct((B, 1, H, D), jnp.float32),
        in_specs=[
            pl.BlockSpec(memory_space=pltpu.VMEM),
            pl.BlockSpec(memory_space=pltpu.MemorySpace.HBM),
            pl.BlockSpec(memory_space=pltpu.MemorySpace.HBM),
            pl.BlockSpec(memory_space=pltpu.VMEM),
            pl.BlockSpec(memory_space=pltpu.VMEM),
        ],
        out_specs=pl.BlockSpec(memory_space=pltpu.VMEM),
        scratch_shapes=[
            pltpu.VMEM((3, HL, D + 2, B), jnp.float32),
            pltpu.VMEM((4, HL, D, B), jnp.float32),
            pltpu.VMEM((NSLOT, CK, HL * D), jnp.float32),
            pltpu.VMEM((NSLOT, CK, HL * D), jnp.float32),
            pltpu.SemaphoreType.DMA((NSLOT,)),
            pltpu.SemaphoreType.DMA((NSLOT,)),
            pltpu.SemaphoreType.DMA,
            pltpu.SemaphoreType.DMA,
            pltpu.SemaphoreType.DMA,
            pltpu.SemaphoreType.DMA((3,)),
            pltpu.SemaphoreType.DMA((3,)),
        ],
        compiler_params=pltpu.CompilerParams(collective_id=0),
    )(q_my, Kr, Vr, bt, lens.reshape(B, 1))


# device time: 21410 ns/iter; 3.4313x vs baseline; 3.4313x over previous
import jax
import jax.numpy as jnp
from jax import lax
from jax.experimental import pallas as pl
from jax.experimental.pallas import tpu as pltpu

B, H, D, BS = 8, 8, 128, 16
NB = 512
HL = 2
CP = 64
CK = CP * BS
NSLOT = 4

NEG_INF = -1e30


def kernel(Q, K, V, bt, lens):
    npages = K.shape[0]
    nchunks = npages // CP
    nt = nchunks * HL

    hb = (lax.axis_index("x") * 2 + lax.axis_index("z")) * HL
    q_my = lax.dynamic_slice_in_dim(Q.reshape(B, H, D), hb, HL, axis=1)

    def body(q_ref, k_hbm, v_hbm, bt_ref, lens_ref, out_ref,
             comm_ref, stage_ref, gbuf, kbuf, vbuf, ksem, vsem, gsem,
             send_sem, recv_sem, ag_send, ag_recv):
        my_x = lax.axis_index("x")
        my_y = lax.axis_index("y")
        my_z = lax.axis_index("z")
        my_idx = my_x * 2 + my_z
        hb_ = my_idx * HL

        y_partner = (my_x, 1 - my_y, my_z)
        ag_partners = [
            ((1 - my_x, my_y, my_z), (1 - my_x) * 2 + my_z, 0),
            ((my_x, my_y, 1 - my_z), my_x * 2 + (1 - my_z), 1),
            ((1 - my_x, my_y, 1 - my_z), (1 - my_x) * 2 + (1 - my_z), 2),
        ]

        barrier = pltpu.get_barrier_semaphore()
        for dev in [y_partner] + [p[0] for p in ag_partners]:
            pl.semaphore_signal(barrier, inc=1, device_id=dev,
                                device_id_type=pl.DeviceIdType.MESH)
        pl.semaphore_wait(barrier, 4)

        def start_copy(t):
            c, j = divmod(t, HL)
            slot = t % NSLOT
            kc = pltpu.make_async_copy(
                k_hbm.at[pl.ds(c * CP, CP), :, hb_ + j, :],
                kbuf.at[slot], ksem.at[slot])
            vc = pltpu.make_async_copy(
                v_hbm.at[pl.ds(c * CP, CP), :, hb_ + j, :],
                vbuf.at[slot], vsem.at[slot])
            kc.start()
            vc.start()
            return kc, vc

        copies = [start_copy(t) for t in range(NSLOT - 1)]

        page_off = my_y * npages
        bt_v = bt_ref[:, :]
        lens_v = lens_ref[:, :]
        slot_i = lax.broadcasted_iota(jnp.int32, (B, NB, 1), 1)
        valid = slot_i < lens_v[:, :, None]
        pids = lax.broadcasted_iota(jnp.int32, (B, NB, npages), 2) + page_off
        match = jnp.logical_and(bt_v[:, :, None] == pids, valid)
        counts = jnp.sum(match.astype(jnp.float32), axis=1)
        w_all = jnp.repeat(counts, BS, axis=1)

        scale = D ** -0.5
        m = [None] * HL
        l = [None] * HL
        acc = [None] * HL
        for t in range(nt):
            if t + NSLOT - 1 < nt:
                copies.append(start_copy(t + NSLOT - 1))
            c, j = divmod(t, HL)
            slot = t % NSLOT
            kc, vc = copies[t]
            kc.wait()
            vc.wait()
            kh = kbuf[slot]
            vh = vbuf[slot]
            qh = q_ref[:, j, :]
            w = w_all[:, c * CK:(c + 1) * CK]
            s = lax.dot_general(
                qh, kh.reshape(CK, D), (((1,), (1,)), ((), ())),
                preferred_element_type=jnp.float32) * scale
            m_c = jnp.max(s, axis=1, keepdims=True)
            if m[j] is None:
                m_new = m_c
                alpha = None
            else:
                m_new = jnp.maximum(m[j], m_c)
                alpha = jnp.exp(m[j] - m_new)
            e = jnp.exp(s - m_new) * w
            pv = lax.dot_general(
                e, vh.reshape(CK, D), (((1,), (0,)), ((), ())),
                preferred_element_type=jnp.float32)
            l_c = jnp.sum(e, axis=1, keepdims=True)
            if alpha is None:
                acc[j] = pv
                l[j] = l_c
            else:
                acc[j] = alpha * acc[j] + pv
                l[j] = alpha * l[j] + l_c
            m[j] = m_new

        accs = jnp.stack(acc, axis=1)
        ms = jnp.concatenate(m, axis=1)
        ls = jnp.concatenate(l, axis=1)

        comm_ref[0] = jnp.concatenate(
            [accs,
             jnp.broadcast_to(ms[:, :, None], (B, HL, D)),
             jnp.broadcast_to(ls[:, :, None], (B, HL, D))], axis=2)

        rdma = pltpu.make_async_remote_copy(
            src_ref=comm_ref.at[0],
            dst_ref=comm_ref.at[1],
            send_sem=send_sem,
            recv_sem=recv_sem,
            device_id=y_partner,
            device_id_type=pl.DeviceIdType.MESH,
        )
        rdma.start()
        rdma.wait()

        r = comm_ref[1]
        acc2 = r[:, :, :D]
        m2 = r[:, :, D]
        l2 = r[:, :, 2 * D]

        m12 = jnp.maximum(ms, m2)
        a1 = jnp.exp(ms - m12)
        a2 = jnp.exp(m2 - m12)
        lsum = a1 * ls + a2 * l2
        o_my = (a1[:, :, None] * accs + a2[:, :, None] * acc2) \
            / lsum[:, :, None]

        stage_ref[...] = o_my
        local = pltpu.make_async_copy(stage_ref, gbuf.at[my_idx], gsem)
        local.start()
        sends = []
        recvs = []
        for dev, pidx, ds in ag_partners:
            snd = pltpu.make_async_remote_copy(
                src_ref=stage_ref,
                dst_ref=gbuf.at[my_idx],
                send_sem=ag_send.at[ds],
                recv_sem=ag_recv.at[ds],
                device_id=dev,
                device_id_type=pl.DeviceIdType.MESH,
            )
            snd.start()
            sends.append(snd)
            recvs.append(pltpu.make_async_remote_copy(
                src_ref=stage_ref,
                dst_ref=gbuf.at[pidx],
                send_sem=ag_send.at[ds],
                recv_sem=ag_recv.at[ds],
                device_id=dev,
                device_id_type=pl.DeviceIdType.MESH,
            ))
        local.wait()
        for snd in sends:
            snd.wait_send()
        for rcv in recvs:
            rcv.wait_recv()

        g = gbuf[...]
        out_ref[:, :, :, :] = jnp.transpose(
            g, (1, 0, 2, 3)).reshape(B, H, D)[:, None, :, :]

    return pl.pallas_call(
        body,
        out_shape=jax.ShapeDtypeStruct((B, 1, H, D), jnp.float32),
        in_specs=[
            pl.BlockSpec(memory_space=pltpu.VMEM),
            pl.BlockSpec(memory_space=pltpu.MemorySpace.HBM),
            pl.BlockSpec(memory_space=pltpu.MemorySpace.HBM),
            pl.BlockSpec(memory_space=pltpu.VMEM),
            pl.BlockSpec(memory_space=pltpu.VMEM),
        ],
        out_specs=pl.BlockSpec(memory_space=pltpu.VMEM),
        scratch_shapes=[
            pltpu.VMEM((2, B, HL, 3 * D), jnp.float32),
            pltpu.VMEM((B, HL, D), jnp.float32),
            pltpu.VMEM((4, B, HL, D), jnp.float32),
            pltpu.VMEM((NSLOT, CP, BS, D), jnp.float32),
            pltpu.VMEM((NSLOT, CP, BS, D), jnp.float32),
            pltpu.SemaphoreType.DMA((NSLOT,)),
            pltpu.SemaphoreType.DMA((NSLOT,)),
            pltpu.SemaphoreType.DMA,
            pltpu.SemaphoreType.DMA,
            pltpu.SemaphoreType.DMA,
            pltpu.SemaphoreType.DMA((3,)),
            pltpu.SemaphoreType.DMA((3,)),
        ],
        compiler_params=pltpu.CompilerParams(collective_id=0),
    )(q_my, K, V, bt, lens.reshape(B, 1))


# device time: 18993 ns/iter; 3.8680x vs baseline; 1.1273x over previous
import jax
import jax.numpy as jnp
from jax import lax
from jax.experimental import pallas as pl
from jax.experimental.pallas import tpu as pltpu

B, H, D, BS = 8, 8, 128, 16
NB = 512
HL = 2
CP = 64
CK = CP * BS
NSLOT = 4

NEG_INF = -1e30


def kernel(Q, K, V, bt, lens):
    npages = K.shape[0]
    nkeys = npages * BS
    nchunks = npages // CP

    hb = (lax.axis_index("x") * 2 + lax.axis_index("z")) * HL
    q_my = lax.dynamic_slice_in_dim(Q.reshape(B, H, D), hb, HL, axis=1)

    def body(q_ref, k_hbm, v_hbm, bt_ref, lens_ref, out_ref,
             comm_ref, stage_ref, gbuf, kbuf, vbuf, ksem, vsem, gsem,
             send_sem, recv_sem, ag_send, ag_recv):
        my_x = lax.axis_index("x")
        my_y = lax.axis_index("y")
        my_z = lax.axis_index("z")
        my_idx = my_x * 2 + my_z
        hb_ = my_idx * HL

        y_partner = (my_x, 1 - my_y, my_z)
        ag_partners = [
            ((1 - my_x, my_y, my_z), (1 - my_x) * 2 + my_z, 0),
            ((my_x, my_y, 1 - my_z), my_x * 2 + (1 - my_z), 1),
            ((1 - my_x, my_y, 1 - my_z), (1 - my_x) * 2 + (1 - my_z), 2),
        ]

        barrier = pltpu.get_barrier_semaphore()
        for dev in [y_partner] + [p[0] for p in ag_partners]:
            pl.semaphore_signal(barrier, inc=1, device_id=dev,
                                device_id_type=pl.DeviceIdType.MESH)
        pl.semaphore_wait(barrier, 4)

        kr = k_hbm.reshape(nkeys, H * D)
        vr = v_hbm.reshape(nkeys, H * D)

        def start_copy(c):
            slot = c % NSLOT
            kc = pltpu.make_async_copy(
                kr.at[pl.ds(c * CK, CK), pl.ds(hb_ * D, HL * D)],
                kbuf.at[slot], ksem.at[slot])
            vc = pltpu.make_async_copy(
                vr.at[pl.ds(c * CK, CK), pl.ds(hb_ * D, HL * D)],
                vbuf.at[slot], vsem.at[slot])
            kc.start()
            vc.start()
            return kc, vc

        copies = [start_copy(t) for t in range(NSLOT - 1)]

        page_off = my_y * npages
        bt_v = bt_ref[:, :]
        lens_v = lens_ref[:, :]
        slot_i = lax.broadcasted_iota(jnp.int32, (B, NB, 1), 1)
        valid = slot_i < lens_v[:, :, None]
        pids = lax.broadcasted_iota(jnp.int32, (B, NB, npages), 2) + page_off
        match = jnp.logical_and(bt_v[:, :, None] == pids, valid)
        counts = jnp.sum(match.astype(jnp.float32), axis=1)
        w_all = jnp.repeat(counts, BS, axis=1)

        scale = D ** -0.5
        m = [None] * HL
        l = [None] * HL
        acc = [None] * HL
        for c in range(nchunks):
            if c + NSLOT - 1 < nchunks:
                copies.append(start_copy(c + NSLOT - 1))
            slot = c % NSLOT
            kc, vc = copies[c]
            kc.wait()
            vc.wait()
            w = w_all[:, c * CK:(c + 1) * CK]
            for j in range(HL):
                kh = kbuf[slot][:, j * D:(j + 1) * D]
                vh = vbuf[slot][:, j * D:(j + 1) * D]
                qh = q_ref[:, j, :]
                s = lax.dot_general(
                    qh, kh, (((1,), (1,)), ((), ())),
                    preferred_element_type=jnp.float32) * scale
                m_c = jnp.max(s, axis=1, keepdims=True)
                if m[j] is None:
                    m_new = m_c
                    alpha = None
                else:
                    m_new = jnp.maximum(m[j], m_c)
                    alpha = jnp.exp(m[j] - m_new)
                e = jnp.exp(s - m_new) * w
                pv = lax.dot_general(
                    e, vh, (((1,), (0,)), ((), ())),
                    preferred_element_type=jnp.float32)
                l_c = jnp.sum(e, axis=1, keepdims=True)
                if alpha is None:
                    acc[j] = pv
                    l[j] = l_c
                else:
                    acc[j] = alpha * acc[j] + pv
                    l[j] = alpha * l[j] + l_c
                m[j] = m_new

        accs = jnp.stack(acc, axis=1)
        ms = jnp.concatenate(m, axis=1)
        ls = jnp.concatenate(l, axis=1)

        comm_ref[0] = jnp.concatenate(
            [accs,
             jnp.broadcast_to(ms[:, :, None], (B, HL, D)),
             jnp.broadcast_to(ls[:, :, None], (B, HL, D))], axis=2)

        rdma = pltpu.make_async_remote_copy(
            src_ref=comm_ref.at[0],
            dst_ref=comm_ref.at[1],
            send_sem=send_sem,
            recv_sem=recv_sem,
            device_id=y_partner,
            device_id_type=pl.DeviceIdType.MESH,
        )
        rdma.start()
        rdma.wait()

        r = comm_ref[1]
        acc2 = r[:, :, :D]
        m2 = r[:, :, D]
        l2 = r[:, :, 2 * D]

        m12 = jnp.maximum(ms, m2)
        a1 = jnp.exp(ms - m12)
        a2 = jnp.exp(m2 - m12)
        lsum = a1 * ls + a2 * l2
        o_my = (a1[:, :, None] * accs + a2[:, :, None] * acc2) \
            / lsum[:, :, None]

        stage_ref[...] = o_my
        local = pltpu.make_async_copy(stage_ref, gbuf.at[my_idx], gsem)
        local.start()
        sends = []
        recvs = []
        for dev, pidx, ds in ag_partners:
            snd = pltpu.make_async_remote_copy(
                src_ref=stage_ref,
                dst_ref=gbuf.at[my_idx],
                send_sem=ag_send.at[ds],
                recv_sem=ag_recv.at[ds],
                device_id=dev,
                device_id_type=pl.DeviceIdType.MESH,
            )
            snd.start()
            sends.append(snd)
            recvs.append(pltpu.make_async_remote_copy(
                src_ref=stage_ref,
                dst_ref=gbuf.at[pidx],
                send_sem=ag_send.at[ds],
                recv_sem=ag_recv.at[ds],
                device_id=dev,
                device_id_type=pl.DeviceIdType.MESH,
            ))
        local.wait()
        for snd in sends:
            snd.wait_send()
        for rcv in recvs:
            rcv.wait_recv()

        g = gbuf[...]
        out_ref[:, :, :, :] = jnp.transpose(
            g, (1, 0, 2, 3)).reshape(B, H, D)[:, None, :, :]

    return pl.pallas_call(
        body,
        out_shape=jax.ShapeDtypeStruct((B, 1, H, D), jnp.float32),
        in_specs=[
            pl.BlockSpec(memory_space=pltpu.VMEM),
            pl.BlockSpec(memory_space=pltpu.MemorySpace.HBM),
            pl.BlockSpec(memory_space=pltpu.MemorySpace.HBM),
            pl.BlockSpec(memory_space=pltpu.VMEM),
            pl.BlockSpec(memory_space=pltpu.VMEM),
        ],
        out_specs=pl.BlockSpec(memory_space=pltpu.VMEM),
        scratch_shapes=[
            pltpu.VMEM((2, B, HL, 3 * D), jnp.float32),
            pltpu.VMEM((B, HL, D), jnp.float32),
            pltpu.VMEM((4, B, HL, D), jnp.float32),
            pltpu.VMEM((NSLOT, CK, HL * D), jnp.float32),
            pltpu.VMEM((NSLOT, CK, HL * D), jnp.float32),
            pltpu.SemaphoreType.DMA((NSLOT,)),
            pltpu.SemaphoreType.DMA((NSLOT,)),
            pltpu.SemaphoreType.DMA,
            pltpu.SemaphoreType.DMA,
            pltpu.SemaphoreType.DMA,
            pltpu.SemaphoreType.DMA((3,)),
            pltpu.SemaphoreType.DMA((3,)),
        ],
        compiler_params=pltpu.CompilerParams(collective_id=0),
    )(q_my, K, V, bt, lens.reshape(B, 1))
